# baseline (speedup 1.0000x reference)
import jax
import jax.numpy as jnp
from jax.experimental import pallas as pl
from jax.experimental.pallas import tpu as pltpu

_INV_SQRT2 = 0.7071067811865476

_ERF_A = (0.254829592, -0.284496736, 1.421413741, -1.453152027, 1.061405429)
_ERF_P = 0.3275911


def _erf_approx(v):
    a1, a2, a3, a4, a5 = _ERF_A
    s = jnp.where(v < 0.0, -1.0, 1.0)
    av = jnp.abs(v)
    t = 1.0 / (1.0 + _ERF_P * av)
    poly = t * (a1 + t * (a2 + t * (a3 + t * (a4 + t * a5))))
    return s * (1.0 - poly * jnp.exp(-av * av))


def _gelu(v):
    return 0.5 * v * (1.0 + _erf_approx(v * _INV_SQRT2))


def _se_kernel(x_ref, w1_ref, w2_ref, ones_ref, o_ref):
    xs = x_ref[0]
    hw = xs.shape[-1]
    pooled = jnp.dot(xs, ones_ref[...],
                     preferred_element_type=jnp.float32) * (1.0 / hw)
    h = jnp.dot(w1_ref[...], pooled,
                preferred_element_type=jnp.float32)
    h = _gelu(h)
    g = jnp.dot(w2_ref[...], h,
                preferred_element_type=jnp.float32)
    gate = 1.0 / (1.0 + jnp.exp(-g))
    o_ref[0] = xs * gate


def kernel(x_nchw, w1, w2):
    N, C, H, W = x_nchw.shape
    HW = H * W
    Cr = w1.shape[0]
    x = x_nchw.reshape(N, C, HW)
    ones = jnp.ones((HW, 1), dtype=jnp.float32)

    out = pl.pallas_call(
        _se_kernel,
        out_shape=jax.ShapeDtypeStruct((N, C, HW), x_nchw.dtype),
        grid=(N,),
        in_specs=[
            pl.BlockSpec((1, C, HW), lambda b: (b, 0, 0)),
            pl.BlockSpec((Cr, C), lambda b: (0, 0)),
            pl.BlockSpec((C, Cr), lambda b: (0, 0)),
            pl.BlockSpec((HW, 1), lambda b: (0, 0)),
        ],
        out_specs=pl.BlockSpec((1, C, HW), lambda b: (b, 0, 0)),
        compiler_params=pltpu.CompilerParams(
            dimension_semantics=("parallel",),
            vmem_limit_bytes=64 * 1024 * 1024,
        ),
    )(x, w1, w2, ones)

    return out.reshape(N, C, H, W)


# speedup vs baseline: 3.4629x; 3.4629x over previous
import jax
import jax.numpy as jnp
from jax import lax
from jax.experimental import pallas as pl
from jax.experimental.pallas import tpu as pltpu

_INV_SQRT2 = 0.7071067811865476

_ERF_A = (0.254829592, -0.284496736, 1.421413741, -1.453152027, 1.061405429)
_ERF_P = 0.3275911


def _erf_approx(v):
    a1, a2, a3, a4, a5 = _ERF_A
    s = jnp.where(v < 0.0, -1.0, 1.0)
    av = jnp.abs(v)
    t = 1.0 / (1.0 + _ERF_P * av)
    poly = t * (a1 + t * (a2 + t * (a3 + t * (a4 + t * a5))))
    return s * (1.0 - poly * jnp.exp(-av * av))


def _gelu(v):
    return 0.5 * v * (1.0 + _erf_approx(v * _INV_SQRT2))


def _se_nhwc_kernel(x_ref, w1_ref, w2t_ref, o_ref):
    xs = x_ref[0]
    hw = xs.shape[0]
    pooled = jnp.sum(xs, axis=0, keepdims=True) * (1.0 / hw)
    h = lax.dot_general(pooled, w1_ref[...],
                        (((1,), (1,)), ((), ())),
                        preferred_element_type=jnp.float32)
    h = _gelu(h)
    g = lax.dot_general(h, w2t_ref[...],
                        (((1,), (0,)), ((), ())),
                        preferred_element_type=jnp.float32)
    gate = 1.0 / (1.0 + jnp.exp(-g))
    o_ref[0] = xs * gate


def kernel(x_nchw, w1, w2):
    N, C, H, W = x_nchw.shape
    HW = H * W
    Cr = w1.shape[0]
    x = jnp.transpose(x_nchw, (0, 2, 3, 1)).reshape(N, HW, C)
    w2t = w2.T

    out = pl.pallas_call(
        _se_nhwc_kernel,
        out_shape=jax.ShapeDtypeStruct((N, HW, C), x_nchw.dtype),
        grid=(N,),
        in_specs=[
            pl.BlockSpec((1, HW, C), lambda b: (b, 0, 0)),
            pl.BlockSpec((Cr, C), lambda b: (0, 0)),
            pl.BlockSpec((Cr, C), lambda b: (0, 0)),
        ],
        out_specs=pl.BlockSpec((1, HW, C), lambda b: (b, 0, 0)),
        compiler_params=pltpu.CompilerParams(
            dimension_semantics=("parallel",),
            vmem_limit_bytes=64 * 1024 * 1024,
        ),
    )(x, w1, w2t)

    return out.reshape(N, H, W, C).transpose(0, 3, 1, 2)


# speedup vs baseline: 4.1052x; 1.1855x over previous
import jax
import jax.numpy as jnp
from jax import lax
from jax.experimental import pallas as pl
from jax.experimental.pallas import tpu as pltpu

_INV_SQRT2 = 0.7071067811865476

_ERF_A = (0.254829592, -0.284496736, 1.421413741, -1.453152027, 1.061405429)
_ERF_P = 0.3275911


def _erf_approx(v):
    a1, a2, a3, a4, a5 = _ERF_A
    s = jnp.where(v < 0.0, -1.0, 1.0)
    av = jnp.abs(v)
    t = 1.0 / (1.0 + _ERF_P * av)
    poly = t * (a1 + t * (a2 + t * (a3 + t * (a4 + t * a5))))
    return s * (1.0 - poly * jnp.exp(-av * av))


def _gelu(v):
    return 0.5 * v * (1.0 + _erf_approx(v * _INV_SQRT2))


def _se_nhwc_kernel(x_ref, w1_ref, w2t_ref, o_ref):
    nb = x_ref.shape[0]
    hw = x_ref.shape[1]
    pooled = jnp.concatenate(
        [jnp.sum(x_ref[i], axis=0, keepdims=True) for i in range(nb)], axis=0
    ) * (1.0 / hw)
    h = lax.dot_general(pooled, w1_ref[...],
                        (((1,), (1,)), ((), ())),
                        preferred_element_type=jnp.float32)
    h = _gelu(h)
    g = lax.dot_general(h, w2t_ref[...],
                        (((1,), (0,)), ((), ())),
                        preferred_element_type=jnp.float32)
    gate = 1.0 / (1.0 + jnp.exp(-g))
    for i in range(nb):
        o_ref[i] = x_ref[i] * gate[i:i + 1]


def kernel(x_nchw, w1, w2):
    N, C, H, W = x_nchw.shape
    HW = H * W
    Cr = w1.shape[0]
    x = jnp.transpose(x_nchw, (0, 2, 3, 1)).reshape(N, HW, C)
    w2t = w2.T

    NB = 2
    out = pl.pallas_call(
        _se_nhwc_kernel,
        out_shape=jax.ShapeDtypeStruct((N, HW, C), x_nchw.dtype),
        grid=(N // NB,),
        in_specs=[
            pl.BlockSpec((NB, HW, C), lambda b: (b, 0, 0)),
            pl.BlockSpec((Cr, C), lambda b: (0, 0)),
            pl.BlockSpec((Cr, C), lambda b: (0, 0)),
        ],
        out_specs=pl.BlockSpec((NB, HW, C), lambda b: (b, 0, 0)),
        compiler_params=pltpu.CompilerParams(
            dimension_semantics=("parallel",),
            vmem_limit_bytes=64 * 1024 * 1024,
        ),
    )(x, w1, w2t)

    return out.reshape(N, H, W, C).transpose(0, 3, 1, 2)


# speedup vs baseline: 4.2250x; 1.0292x over previous
import jax
import jax.numpy as jnp
from jax import lax
from jax.experimental import pallas as pl
from jax.experimental.pallas import tpu as pltpu

_INV_SQRT2 = 0.7071067811865476

_ERF_A = (0.254829592, -0.284496736, 1.421413741, -1.453152027, 1.061405429)
_ERF_P = 0.3275911


def _erf_approx(v):
    a1, a2, a3, a4, a5 = _ERF_A
    s = jnp.where(v < 0.0, -1.0, 1.0)
    av = jnp.abs(v)
    t = 1.0 / (1.0 + _ERF_P * av)
    poly = t * (a1 + t * (a2 + t * (a3 + t * (a4 + t * a5))))
    return s * (1.0 - poly * jnp.exp(-av * av))


def _gelu(v):
    return 0.5 * v * (1.0 + _erf_approx(v * _INV_SQRT2))


def _se_nhwc_kernel(x_ref, w1_ref, w2t_ref, o_ref):
    nb = x_ref.shape[0]
    hw = x_ref.shape[1]
    pooled = jnp.concatenate(
        [jnp.sum(x_ref[i], axis=0, keepdims=True) for i in range(nb)], axis=0
    ) * (1.0 / hw)
    h = lax.dot_general(pooled, w1_ref[...],
                        (((1,), (1,)), ((), ())),
                        preferred_element_type=jnp.float32)
    h = _gelu(h)
    g = lax.dot_general(h, w2t_ref[...],
                        (((1,), (0,)), ((), ())),
                        preferred_element_type=jnp.float32)
    gate = 1.0 / (1.0 + jnp.exp(-g))
    for i in range(nb):
        o_ref[i] = x_ref[i] * gate[i:i + 1]


def kernel(x_nchw, w1, w2):
    N, C, H, W = x_nchw.shape
    HW = H * W
    Cr = w1.shape[0]
    x = jnp.transpose(x_nchw, (0, 2, 3, 1)).reshape(N, HW, C)
    w2t = w2.T

    NB = 4
    out = pl.pallas_call(
        _se_nhwc_kernel,
        out_shape=jax.ShapeDtypeStruct((N, HW, C), x_nchw.dtype),
        grid=(N // NB,),
        in_specs=[
            pl.BlockSpec((NB, HW, C), lambda b: (b, 0, 0)),
            pl.BlockSpec((Cr, C), lambda b: (0, 0)),
            pl.BlockSpec((Cr, C), lambda b: (0, 0)),
        ],
        out_specs=pl.BlockSpec((NB, HW, C), lambda b: (b, 0, 0)),
        compiler_params=pltpu.CompilerParams(
            dimension_semantics=("parallel",),
            vmem_limit_bytes=64 * 1024 * 1024,
        ),
    )(x, w1, w2t)

    return out.reshape(N, H, W, C).transpose(0, 3, 1, 2)
